# baseline (speedup 1.0000x reference)
import functools

import jax
import jax.numpy as jnp
from jax import lax
from jax.experimental import pallas as pl
from jax.experimental.pallas import tpu as pltpu
from jax.experimental.pallas import tpu_sc as plsc

N = 10000
E = 320000
NPAD = 10240
CHUNK = 32
NSLOT = 8
LEAD = 6
TILES = 16
EPAD = 327680
EPT = EPAD // TILES
EPT0 = EPAD // 32
ZROWS = NPAD // TILES
DH = 128
SBE = 2048
SBC = SBE // CHUNK


def _edge_pipeline(x_hbm, agg_s, src_v, dst_v, rows_v, semg, sems):
    def gather(k, slot):
        off = pl.multiple_of(k * CHUNK, CHUNK)
        pltpu.async_copy(x_hbm.at[src_v.at[pl.ds(off, CHUNK)]],
                         rows_v.at[slot], semg)

    def wait_gather():
        pltpu.make_async_copy(x_hbm.at[pl.ds(0, CHUNK)], rows_v.at[0],
                              semg).wait()

    def wait_scatter():
        pltpu.make_async_copy(x_hbm.at[pl.ds(0, CHUNK)], rows_v.at[0],
                              sems).wait()

    for k in range(LEAD):
        gather(k, k)

    def body(j, carry):
        slot = lax.rem(j, NSLOT)
        wait_gather()
        pltpu.async_copy(rows_v.at[slot], agg_s.at[dst_v.at[j]], sems,
                         add=True)

        @pl.when(j >= 2)
        def _():
            wait_scatter()

        @pl.when(j + LEAD < SBC)
        def _():
            gather(j + LEAD, lax.rem(j + LEAD, NSLOT))

        return carry

    lax.fori_loop(0, SBC, body, 0)
    wait_scatter()
    wait_scatter()


def _sc_scatter_planes():
    mesh = plsc.VectorSubcoreMesh(core_axis_name="c", subcore_axis_name="s")

    @functools.partial(
        pl.kernel,
        mesh=mesh,
        out_type=jax.ShapeDtypeStruct((2, NPAD, DH), jnp.float32),
        scratch_types=[
            pltpu.VMEM((SBE,), jnp.int32),
            pltpu.VMEM((SBC, CHUNK), jnp.int32),
            pltpu.VMEM((NSLOT, CHUNK, DH), jnp.float32),
            pltpu.VMEM_SHARED((NPAD, DH), jnp.float32),
            pltpu.SemaphoreType.DMA,
            pltpu.SemaphoreType.DMA,
        ],
    )
    def sc_scatter(xcat, srcs, dsts, zeros, out,
                   src_v, dst_v, rows_v, agg_s, semg, sems):
        c = lax.axis_index("c")
        s = lax.axis_index("s")
        pltpu.sync_copy(zeros, agg_s.at[pl.ds(s * ZROWS, ZROWS)])
        plsc.subcore_barrier()

        def outer(g, carry):
            eoff = pl.multiple_of(s * EPT + g * SBE, CHUNK)
            roff = pl.multiple_of(s * (EPT // CHUNK) + g * SBC, 8)
            pltpu.sync_copy(srcs.at[c, pl.ds(eoff, SBE)], src_v)
            pltpu.sync_copy(dsts.at[pl.ds(roff, SBC)], dst_v)
            _edge_pipeline(xcat, agg_s, src_v, dst_v, rows_v, semg, sems)
            return carry

        lax.fori_loop(0, EPT // SBE, outer, 0)
        plsc.subcore_barrier()
        pltpu.sync_copy(agg_s.at[pl.ds(s * ZROWS, ZROWS)],
                        out.at[c, pl.ds(s * ZROWS, ZROWS)])

    return sc_scatter


def _sc_scatter_l0():
    mesh = plsc.VectorSubcoreMesh(core_axis_name="c", subcore_axis_name="s")

    @functools.partial(
        pl.kernel,
        mesh=mesh,
        out_type=jax.ShapeDtypeStruct((2, NPAD, DH), jnp.float32),
        scratch_types=[
            pltpu.VMEM((SBE,), jnp.int32),
            pltpu.VMEM((SBC, CHUNK), jnp.int32),
            pltpu.VMEM((NSLOT, CHUNK, DH), jnp.float32),
            pltpu.VMEM_SHARED((NPAD, DH), jnp.float32),
            pltpu.SemaphoreType.DMA,
            pltpu.SemaphoreType.DMA,
        ],
    )
    def sc_scatter(xcat, srcs, dsts, zeros, out,
                   src_v, dst_v, rows_v, agg_s, semg, sems):
        c = lax.axis_index("c")
        s = lax.axis_index("s")
        pltpu.sync_copy(zeros, agg_s.at[pl.ds(s * ZROWS, ZROWS)])
        plsc.subcore_barrier()

        def outer(g, carry):
            eoff = pl.multiple_of(s * EPT + g * SBE, CHUNK)
            roff = pl.multiple_of(s * (EPT // CHUNK) + g * SBC, 8)
            pltpu.sync_copy(srcs.at[c, pl.ds(eoff, SBE)], src_v)
            pltpu.sync_copy(dsts.at[c, pl.ds(roff, SBC)], dst_v)
            _edge_pipeline(xcat, agg_s, src_v, dst_v, rows_v, semg, sems)
            return carry

        lax.fori_loop(0, EPT // SBE, outer, 0)
        plsc.subcore_barrier()
        pltpu.sync_copy(agg_s.at[pl.ds(s * ZROWS, ZROWS)],
                        out.at[c, pl.ds(s * ZROWS, ZROWS)])

    return sc_scatter


def _l0_body(x_ref, a_ref, w_ref, b_ref, s_ref, o_ref):
    sc = s_ref[0, 0]
    s0 = sc * x_ref[...] + a_ref[...]
    acc = lax.dot_general(s0, w_ref[...], (((1,), (1,)), ((), ())),
                          preferred_element_type=jnp.float32)
    acc = jnp.maximum(acc + b_ref[...], 0.0)
    o_ref[0] = acc[:, :128]
    o_ref[1] = acc[:, 128:]


def _tc_l0(x, agg, w, b, scale):
    r = 1280
    return pl.pallas_call(
        _l0_body,
        grid=(NPAD // r,),
        in_specs=[
            pl.BlockSpec((r, 128), lambda i: (i, 0)),
            pl.BlockSpec((r, 128), lambda i: (i, 0)),
            pl.BlockSpec((256, 128), lambda i: (0, 0)),
            pl.BlockSpec((1, 256), lambda i: (0, 0)),
            pl.BlockSpec(memory_space=pltpu.SMEM),
        ],
        out_specs=pl.BlockSpec((2, r, 128), lambda i: (0, i, 0)),
        out_shape=jax.ShapeDtypeStruct((2, NPAD, 128), jnp.float32),
    )(x, agg, w, b.reshape(1, 256), scale)


def _mid_body(x_ref, a_ref, w_ref, b_ref, s_ref, o_ref):
    sc = s_ref[0, 0]
    s0 = sc * x_ref[0] + a_ref[0]
    s1 = sc * x_ref[1] + a_ref[1]
    w = w_ref[...]
    acc = lax.dot_general(s0, w[:, :128], (((1,), (1,)), ((), ())),
                          preferred_element_type=jnp.float32)
    acc = acc + lax.dot_general(s1, w[:, 128:], (((1,), (1,)), ((), ())),
                                preferred_element_type=jnp.float32)
    acc = jnp.maximum(acc + b_ref[...], 0.0)
    o_ref[0] = acc[:, :128]
    o_ref[1] = acc[:, 128:]


def _tc_mid(x2, agg2, w, b, scale):
    r = 1280
    return pl.pallas_call(
        _mid_body,
        grid=(NPAD // r,),
        in_specs=[
            pl.BlockSpec((2, r, 128), lambda i: (0, i, 0)),
            pl.BlockSpec((2, r, 128), lambda i: (0, i, 0)),
            pl.BlockSpec((256, 256), lambda i: (0, 0)),
            pl.BlockSpec((1, 256), lambda i: (0, 0)),
            pl.BlockSpec(memory_space=pltpu.SMEM),
        ],
        out_specs=pl.BlockSpec((2, r, 128), lambda i: (0, i, 0)),
        out_shape=jax.ShapeDtypeStruct((2, NPAD, 128), jnp.float32),
    )(x2, agg2, w, b.reshape(1, 256), scale)


def _last_body(do_abs, x_ref, a_ref, w_ref, b_ref, s_ref, hw_ref, hb_ref, o_ref):
    sc = s_ref[0, 0]
    s0 = sc * x_ref[0] + a_ref[0]
    s1 = sc * x_ref[1] + a_ref[1]
    w = w_ref[...]
    acc = lax.dot_general(s0, w[:, :128], (((1,), (1,)), ((), ())),
                          preferred_element_type=jnp.float32)
    acc = acc + lax.dot_general(s1, w[:, 128:], (((1,), (1,)), ((), ())),
                                preferred_element_type=jnp.float32)
    acc = jnp.maximum(acc + b_ref[...], 0.0)
    ho = lax.dot_general(acc, hw_ref[...], (((1,), (1,)), ((), ())),
                         preferred_element_type=jnp.float32) + hb_ref[...]
    o_ref[...] = jnp.abs(ho) if do_abs else ho


def _tc_last(x2, agg2, w, b, scale, head_w, head_b, do_abs):
    hw = head_w.shape[0]
    hwp = jnp.zeros((128, 256), jnp.float32).at[:hw].set(head_w)
    hbp = jnp.zeros((1, 128), jnp.float32).at[0, :hw].set(head_b)
    r = 1280
    return pl.pallas_call(
        functools.partial(_last_body, do_abs),
        grid=(NPAD // r,),
        in_specs=[
            pl.BlockSpec((2, r, 128), lambda i: (0, i, 0)),
            pl.BlockSpec((2, r, 128), lambda i: (0, i, 0)),
            pl.BlockSpec((256, 256), lambda i: (0, 0)),
            pl.BlockSpec((1, 256), lambda i: (0, 0)),
            pl.BlockSpec(memory_space=pltpu.SMEM),
            pl.BlockSpec((128, 256), lambda i: (0, 0)),
            pl.BlockSpec((1, 128), lambda i: (0, 0)),
        ],
        out_specs=pl.BlockSpec((r, 128), lambda i: (i, 0)),
        out_shape=jax.ShapeDtypeStruct((NPAD, 128), jnp.float32),
    )(x2, agg2, w, b.reshape(1, 256), scale, hwp, hbp)[:, :hw]


def _edge_prep(ei):
    pad = jnp.arange(EPAD - E, dtype=jnp.int32)
    src = ei[0]
    dst = ei[1]
    srcp = jnp.concatenate([src, pad % N])
    srcs2 = jnp.stack([srcp, srcp + NPAD])
    dstp = jnp.concatenate([dst, N + pad % (NPAD - N)])
    return srcp, srcs2, dstp.reshape(EPAD // CHUNK, CHUNK)


def kernel(high_emb, low_emb, high_edge_index, low_edge_index,
           W0, b0, eps0, W1, b1, eps1, W2, b2, eps2,
           high_W, high_b, low_W, low_b, alpha):
    f32 = jnp.float32
    src_h, srcs2_h, dst_h = _edge_prep(high_edge_index)
    src_l, srcs2_l, dst_l = _edge_prep(low_edge_index)
    z128 = jnp.zeros((ZROWS, DH), f32)
    sc_l0 = _sc_scatter_l0()
    sc_pl = _sc_scatter_planes()
    xh = jnp.pad(high_emb, ((0, NPAD - N), (0, 0)))
    xl = jnp.pad(low_emb, ((0, NPAD - N), (0, 0)))
    xcat0 = jnp.concatenate([xh, xl])
    srcs_l0 = jnp.stack([src_h, src_l + NPAD])
    dsts_l0 = jnp.stack([dst_h, dst_l])
    s0 = (1.0 + eps0).reshape(1, 1)
    s1 = (1.0 + eps1).reshape(1, 1)
    s2 = (1.0 + eps2).reshape(1, 1)
    agg0 = sc_l0(xcat0, srcs_l0, dsts_l0, z128)
    x2h = _tc_l0(xh, agg0[0], W0, b0, s0)
    x2l = _tc_l0(xl, agg0[1], W0, b0, s0)
    agg_h = sc_pl(x2h.reshape(2 * NPAD, DH), srcs2_h, dst_h, z128)
    agg_l = sc_pl(x2l.reshape(2 * NPAD, DH), srcs2_l, dst_l, z128)
    x2h = _tc_mid(x2h, agg_h, W1, b1, s1)
    x2l = _tc_mid(x2l, agg_l, W1, b1, s1)
    agg_h = sc_pl(x2h.reshape(2 * NPAD, DH), srcs2_h, dst_h, z128)
    agg_l = sc_pl(x2l.reshape(2 * NPAD, DH), srcs2_l, dst_l, z128)
    h_out = _tc_last(x2h, agg_h, W2, b2, s2, high_W, high_b, False)[:N]
    l_out = _tc_last(x2l, agg_l, W2, b2, s2, low_W, low_b, True)[:N]
    return (h_out, l_out, jax.nn.sigmoid(alpha))


# speedup vs baseline: 1.0946x; 1.0946x over previous
import functools

import jax
import jax.numpy as jnp
from jax import lax
from jax.experimental import pallas as pl
from jax.experimental.pallas import tpu as pltpu
from jax.experimental.pallas import tpu_sc as plsc

N = 10000
E = 320000
NPAD = 10240
CHUNK = 32
NSLOT = 8
LEAD = 6
TILES = 16
EPAD = 327680
EPT = EPAD // TILES
EPT0 = EPAD // 32
ZROWS = NPAD // TILES
DH = 128
SBE = 1024
SBC = SBE // CHUNK
NIB = 3


def _edge_stream(x_hbm, agg_s, stage, wait_stage, nchunks,
                 src_v, dst_v, rows_v, semg, sems):
    nsb = nchunks // SBC

    def gather(k, slot, b):
        off = pl.multiple_of(b * SBE + lax.rem(k, SBC) * CHUNK, CHUNK)
        pltpu.async_copy(x_hbm.at[src_v.at[pl.ds(off, CHUNK)]],
                         rows_v.at[slot], semg)

    def wait_gather():
        pltpu.make_async_copy(x_hbm.at[pl.ds(0, CHUNK)], rows_v.at[0],
                              semg).wait()

    def wait_scatter():
        pltpu.make_async_copy(x_hbm.at[pl.ds(0, CHUNK)], rows_v.at[0],
                              sems).wait()

    stage(0, jnp.int32(0))
    wait_stage(jnp.int32(0))
    stage(1, jnp.int32(1))
    for k in range(LEAD):
        gather(k, k, 0)

    def body(j, carry):
        slot = lax.rem(j, NSLOT)
        b = lax.rem(lax.div(j, SBC), NIB)
        wait_gather()
        pltpu.async_copy(rows_v.at[slot],
                         agg_s.at[dst_v.at[b * SBC + lax.rem(j, SBC)]],
                         sems, add=True)

        @pl.when(j >= 2)
        def _():
            wait_scatter()

        jn = j + LEAD

        @pl.when(jn < nchunks)
        def _():
            g = lax.div(jn, SBC)
            bn = lax.rem(g, NIB)

            @pl.when(lax.rem(jn, SBC) == 0)
            def _():
                wait_stage(bn)

                @pl.when(g + 1 < nsb)
                def _():
                    stage(g + 1, lax.rem(g + 1, NIB))

            gather(jn, lax.rem(jn, NSLOT), bn)

        return carry

    lax.fori_loop(0, nchunks, body, 0)
    wait_scatter()
    wait_scatter()


def _sc_scatter_planes():
    mesh = plsc.VectorSubcoreMesh(core_axis_name="c", subcore_axis_name="s")

    @functools.partial(
        pl.kernel,
        mesh=mesh,
        out_type=jax.ShapeDtypeStruct((2, NPAD, DH), jnp.float32),
        scratch_types=[
            pltpu.VMEM((NIB * SBE,), jnp.int32),
            pltpu.VMEM((NIB * SBC, CHUNK), jnp.int32),
            pltpu.VMEM((NSLOT, CHUNK, DH), jnp.float32),
            pltpu.VMEM_SHARED((NPAD, DH), jnp.float32),
            pltpu.SemaphoreType.DMA,
            pltpu.SemaphoreType.DMA,
            pltpu.SemaphoreType.DMA,
            pltpu.SemaphoreType.DMA,
            pltpu.SemaphoreType.DMA,
        ],
    )
    def sc_scatter(xcat, srcs, dsts, zeros, out,
                   src_v, dst_v, rows_v, agg_s, semg, sems, si0, si1, si2):
        c = lax.axis_index("c")
        s = lax.axis_index("s")
        sis = (si0, si1, si2)
        pltpu.sync_copy(zeros, agg_s.at[pl.ds(s * ZROWS, ZROWS)])
        plsc.subcore_barrier()

        def stage_b(g, bi):
            eoff = pl.multiple_of(s * EPT + g * SBE, CHUNK)
            roff = pl.multiple_of(s * (EPT // CHUNK) + g * SBC, 8)
            pltpu.async_copy(srcs.at[c, pl.ds(eoff, SBE)],
                             src_v.at[pl.ds(bi * SBE, SBE)], sis[bi])
            pltpu.async_copy(dsts.at[pl.ds(roff, SBC)],
                             dst_v.at[pl.ds(bi * SBC, SBC)], sis[bi])

        def stage(g, b):
            for bi in range(NIB):
                @pl.when(b == bi)
                def _(bi=bi):
                    stage_b(g, bi)

        def wait_stage(b):
            for bi in range(NIB):
                @pl.when(b == bi)
                def _(bi=bi):
                    pltpu.make_async_copy(srcs.at[c, pl.ds(0, SBE)],
                                          src_v.at[pl.ds(0, SBE)],
                                          sis[bi]).wait()
                    pltpu.make_async_copy(dsts.at[pl.ds(0, SBC)],
                                          dst_v.at[pl.ds(0, SBC)],
                                          sis[bi]).wait()

        _edge_stream(xcat, agg_s, stage, wait_stage, EPT // CHUNK,
                     src_v, dst_v, rows_v, semg, sems)
        plsc.subcore_barrier()
        pltpu.sync_copy(agg_s.at[pl.ds(s * ZROWS, ZROWS)],
                        out.at[c, pl.ds(s * ZROWS, ZROWS)])

    return sc_scatter


def _sc_scatter_edgesplit():
    mesh = plsc.VectorSubcoreMesh(core_axis_name="c", subcore_axis_name="s")

    @functools.partial(
        pl.kernel,
        mesh=mesh,
        out_type=jax.ShapeDtypeStruct((2, NPAD, DH), jnp.float32),
        scratch_types=[
            pltpu.VMEM((NIB * SBE,), jnp.int32),
            pltpu.VMEM((NIB * SBC, CHUNK), jnp.int32),
            pltpu.VMEM((NSLOT, CHUNK, DH), jnp.float32),
            pltpu.VMEM_SHARED((NPAD, DH), jnp.float32),
            pltpu.SemaphoreType.DMA,
            pltpu.SemaphoreType.DMA,
            pltpu.SemaphoreType.DMA,
            pltpu.SemaphoreType.DMA,
            pltpu.SemaphoreType.DMA,
        ],
    )
    def sc_scatter(x, srcs, dsts, zeros, out,
                   src_v, dst_v, rows_v, agg_s, semg, sems, si0, si1, si2):
        c = lax.axis_index("c")
        s = lax.axis_index("s")
        w = c * TILES + s
        sis = (si0, si1, si2)
        pltpu.sync_copy(zeros, agg_s.at[pl.ds(s * ZROWS, ZROWS)])
        plsc.subcore_barrier()

        def stage_b(g, bi):
            eoff = pl.multiple_of(w * EPT0 + g * SBE, CHUNK)
            roff = pl.multiple_of(w * (EPT0 // CHUNK) + g * SBC, 8)
            pltpu.async_copy(srcs.at[pl.ds(eoff, SBE)],
                             src_v.at[pl.ds(bi * SBE, SBE)], sis[bi])
            pltpu.async_copy(dsts.at[pl.ds(roff, SBC)],
                             dst_v.at[pl.ds(bi * SBC, SBC)], sis[bi])

        def stage(g, b):
            for bi in range(NIB):
                @pl.when(b == bi)
                def _(bi=bi):
                    stage_b(g, bi)

        def wait_stage(b):
            for bi in range(NIB):
                @pl.when(b == bi)
                def _(bi=bi):
                    pltpu.make_async_copy(srcs.at[pl.ds(0, SBE)],
                                          src_v.at[pl.ds(0, SBE)],
                                          sis[bi]).wait()
                    pltpu.make_async_copy(dsts.at[pl.ds(0, SBC)],
                                          dst_v.at[pl.ds(0, SBC)],
                                          sis[bi]).wait()

        _edge_stream(x, agg_s, stage, wait_stage, EPT0 // CHUNK,
                     src_v, dst_v, rows_v, semg, sems)
        plsc.subcore_barrier()
        pltpu.sync_copy(agg_s.at[pl.ds(s * ZROWS, ZROWS)],
                        out.at[c, pl.ds(s * ZROWS, ZROWS)])

    return sc_scatter


def _l0_body(x_ref, a_ref, w_ref, b_ref, s_ref, o_ref):
    sc = s_ref[0, 0]
    s0 = sc * x_ref[...] + a_ref[0] + a_ref[1]
    acc = lax.dot_general(s0, w_ref[...], (((1,), (1,)), ((), ())),
                          preferred_element_type=jnp.float32)
    acc = jnp.maximum(acc + b_ref[...], 0.0)
    o_ref[0] = acc[:, :128]
    o_ref[1] = acc[:, 128:]


def _tc_l0(x, agg2, w, b, scale):
    r = 1280
    return pl.pallas_call(
        _l0_body,
        grid=(NPAD // r,),
        in_specs=[
            pl.BlockSpec((r, 128), lambda i: (i, 0)),
            pl.BlockSpec((2, r, 128), lambda i: (0, i, 0)),
            pl.BlockSpec((256, 128), lambda i: (0, 0)),
            pl.BlockSpec((1, 256), lambda i: (0, 0)),
            pl.BlockSpec(memory_space=pltpu.SMEM),
        ],
        out_specs=pl.BlockSpec((2, r, 128), lambda i: (0, i, 0)),
        out_shape=jax.ShapeDtypeStruct((2, NPAD, 128), jnp.float32),
    )(x, agg2, w, b.reshape(1, 256), scale)


def _mid_body(x_ref, a_ref, w_ref, b_ref, s_ref, o_ref):
    sc = s_ref[0, 0]
    s0 = sc * x_ref[0] + a_ref[0]
    s1 = sc * x_ref[1] + a_ref[1]
    w = w_ref[...]
    acc = lax.dot_general(s0, w[:, :128], (((1,), (1,)), ((), ())),
                          preferred_element_type=jnp.float32)
    acc = acc + lax.dot_general(s1, w[:, 128:], (((1,), (1,)), ((), ())),
                                preferred_element_type=jnp.float32)
    acc = jnp.maximum(acc + b_ref[...], 0.0)
    o_ref[0] = acc[:, :128]
    o_ref[1] = acc[:, 128:]


def _tc_mid(x2, agg2, w, b, scale):
    r = 1280
    return pl.pallas_call(
        _mid_body,
        grid=(NPAD // r,),
        in_specs=[
            pl.BlockSpec((2, r, 128), lambda i: (0, i, 0)),
            pl.BlockSpec((2, r, 128), lambda i: (0, i, 0)),
            pl.BlockSpec((256, 256), lambda i: (0, 0)),
            pl.BlockSpec((1, 256), lambda i: (0, 0)),
            pl.BlockSpec(memory_space=pltpu.SMEM),
        ],
        out_specs=pl.BlockSpec((2, r, 128), lambda i: (0, i, 0)),
        out_shape=jax.ShapeDtypeStruct((2, NPAD, 128), jnp.float32),
    )(x2, agg2, w, b.reshape(1, 256), scale)


def _last_body(do_abs, x_ref, a_ref, w_ref, b_ref, s_ref, hw_ref, hb_ref, o_ref):
    sc = s_ref[0, 0]
    s0 = sc * x_ref[0] + a_ref[0]
    s1 = sc * x_ref[1] + a_ref[1]
    w = w_ref[...]
    acc = lax.dot_general(s0, w[:, :128], (((1,), (1,)), ((), ())),
                          preferred_element_type=jnp.float32)
    acc = acc + lax.dot_general(s1, w[:, 128:], (((1,), (1,)), ((), ())),
                                preferred_element_type=jnp.float32)
    acc = jnp.maximum(acc + b_ref[...], 0.0)
    ho = lax.dot_general(acc, hw_ref[...], (((1,), (1,)), ((), ())),
                         preferred_element_type=jnp.float32) + hb_ref[...]
    o_ref[...] = jnp.abs(ho) if do_abs else ho


def _tc_last(x2, agg2, w, b, scale, head_w, head_b, do_abs):
    hw = head_w.shape[0]
    hwp = jnp.zeros((128, 256), jnp.float32).at[:hw].set(head_w)
    hbp = jnp.zeros((1, 128), jnp.float32).at[0, :hw].set(head_b)
    r = 1280
    return pl.pallas_call(
        functools.partial(_last_body, do_abs),
        grid=(NPAD // r,),
        in_specs=[
            pl.BlockSpec((2, r, 128), lambda i: (0, i, 0)),
            pl.BlockSpec((2, r, 128), lambda i: (0, i, 0)),
            pl.BlockSpec((256, 256), lambda i: (0, 0)),
            pl.BlockSpec((1, 256), lambda i: (0, 0)),
            pl.BlockSpec(memory_space=pltpu.SMEM),
            pl.BlockSpec((128, 256), lambda i: (0, 0)),
            pl.BlockSpec((1, 128), lambda i: (0, 0)),
        ],
        out_specs=pl.BlockSpec((r, 128), lambda i: (i, 0)),
        out_shape=jax.ShapeDtypeStruct((NPAD, 128), jnp.float32),
    )(x2, agg2, w, b.reshape(1, 256), scale, hwp, hbp)[:, :hw]


def _edge_prep(ei):
    pad = jnp.arange(EPAD - E, dtype=jnp.int32)
    src = ei[0]
    dst = ei[1]
    srcp = jnp.concatenate([src, pad % N])
    srcs2 = jnp.stack([srcp, srcp + NPAD])
    dstp = jnp.concatenate([dst, N + pad % (NPAD - N)])
    return srcp, srcs2, dstp.reshape(EPAD // CHUNK, CHUNK)


def kernel(high_emb, low_emb, high_edge_index, low_edge_index,
           W0, b0, eps0, W1, b1, eps1, W2, b2, eps2,
           high_W, high_b, low_W, low_b, alpha):
    f32 = jnp.float32
    src_h, srcs2_h, dst_h = _edge_prep(high_edge_index)
    src_l, srcs2_l, dst_l = _edge_prep(low_edge_index)
    z128 = jnp.zeros((ZROWS, DH), f32)
    sc_es = _sc_scatter_edgesplit()
    sc_pl = _sc_scatter_planes()

    def run_graph(emb, src1, srcs2, dst2, head_w, head_b, do_abs):
        x = jnp.pad(emb, ((0, NPAD - N), (0, 0)))
        agg = sc_es(x, src1, dst2, z128)
        x2 = _tc_l0(x, agg, W0, b0, (1.0 + eps0).reshape(1, 1))
        agg = sc_pl(x2.reshape(2 * NPAD, DH), srcs2, dst2, z128)
        x2 = _tc_mid(x2, agg, W1, b1, (1.0 + eps1).reshape(1, 1))
        agg = sc_pl(x2.reshape(2 * NPAD, DH), srcs2, dst2, z128)
        out = _tc_last(x2, agg, W2, b2, (1.0 + eps2).reshape(1, 1),
                       head_w, head_b, do_abs)
        return out[:N]

    h_out = run_graph(high_emb, src_h, srcs2_h, dst_h, high_W, high_b, False)
    l_out = run_graph(low_emb, src_l, srcs2_l, dst_l, low_W, low_b, True)
    return (h_out, l_out, jax.nn.sigmoid(alpha))
